# baseline (speedup 1.0000x reference)
import functools

import jax
import jax.numpy as jnp
from jax import lax
from jax.experimental import pallas as pl
from jax.experimental.pallas import tpu as pltpu
from jax.experimental.pallas import tpu_sc as plsc

N_NODES = 10000
N_EDGES = 320000
IN_DIM = 128
OUT_DIM = 128
EDGE_DIM = 16

LANES = 16
CHUNK = 128
NCHUNKS = N_EDGES // CHUNK

_BN = 1000
_BE = 3200


def _node_body(h_ref, wu_ref, ww_ref, hu_ref, hw_ref):
    h = h_ref[...]
    hu_ref[...] = jnp.dot(h, wu_ref[...], preferred_element_type=jnp.float32)
    hw_ref[...] = jnp.dot(h, ww_ref[...], preferred_element_type=jnp.float32)


def _node_transform(h, wuT, wwT):
    return pl.pallas_call(
        _node_body,
        grid=(N_NODES // _BN,),
        in_specs=[
            pl.BlockSpec((_BN, IN_DIM), lambda i: (i, 0)),
            pl.BlockSpec((IN_DIM, OUT_DIM), lambda i: (0, 0)),
            pl.BlockSpec((IN_DIM, OUT_DIM), lambda i: (0, 0)),
        ],
        out_specs=[
            pl.BlockSpec((_BN, OUT_DIM), lambda i: (i, 0)),
            pl.BlockSpec((_BN, OUT_DIM), lambda i: (i, 0)),
        ],
        out_shape=[
            jax.ShapeDtypeStruct((N_NODES, OUT_DIM), jnp.float32),
            jax.ShapeDtypeStruct((N_NODES, OUT_DIM), jnp.float32),
        ],
    )(h, wuT, wwT)


def _edge_body(e_hbm, we_ref, me_ref, e_vmem, sem):
    i = pl.program_id(0)
    pltpu.make_async_copy(e_hbm.at[pl.ds(i * _BE, _BE), :], e_vmem,
                          sem).start()
    pltpu.make_async_copy(e_hbm.at[pl.ds(i * _BE, _BE), :], e_vmem,
                          sem).wait()
    me_ref[...] = jnp.dot(e_vmem[...], we_ref[...],
                          preferred_element_type=jnp.float32)


def _edge_transform(e, weT):
    return pl.pallas_call(
        _edge_body,
        grid=(N_EDGES // _BE,),
        in_specs=[
            pl.BlockSpec(memory_space=pl.ANY),
            pl.BlockSpec((EDGE_DIM, OUT_DIM), lambda i: (0, 0)),
        ],
        out_specs=pl.BlockSpec((_BE, OUT_DIM), lambda i: (i, 0)),
        out_shape=jax.ShapeDtypeStruct((N_EDGES, OUT_DIM), jnp.float32),
        scratch_shapes=[
            pltpu.VMEM((_BE, EDGE_DIM), jnp.float32),
            pltpu.SemaphoreType.DMA,
        ],
    )(e, weT)


_NC = 2
_NS = 16
_NW = _NC * _NS


_N_MAIN = NCHUNKS // _NW
_N_TAIL = NCHUNKS - _N_MAIN * _NW
_NBUF = 2
_NACC = 3


def _sc_body(hu_hbm, hw_hbm, src_hbm, tgt_hbm, me_hbm, out_hbm,
             idx_s, idx_t, rows_a, rows_b, acc,
             sem_i0, sem_i1, sem_g0, sem_g1, sem_o0, sem_o1, sem_o2):
    wid = lax.axis_index("s") * _NC + lax.axis_index("c")
    start = wid * _N_MAIN
    sem_i = (sem_i0, sem_i1)
    sem_g = (sem_g0, sem_g1)
    sem_o = (sem_o0, sem_o1, sem_o2)

    def issue_idx(k, b):
        base = (start + k) * CHUNK
        pltpu.async_copy(src_hbm.at[pl.ds(base, CHUNK)], idx_s.at[b], sem_i[b])
        pltpu.async_copy(tgt_hbm.at[pl.ds(base, CHUNK)], idx_t.at[b], sem_i[b])

    def wait_idx(k, b):
        base = (start + k) * CHUNK
        pltpu.make_async_copy(src_hbm.at[pl.ds(base, CHUNK)], idx_s.at[b],
                              sem_i[b]).wait()
        pltpu.make_async_copy(tgt_hbm.at[pl.ds(base, CHUNK)], idx_t.at[b],
                              sem_i[b]).wait()

    def issue_gathers(k, b, a):
        base = (start + k) * CHUNK
        pltpu.async_copy(me_hbm.at[pl.ds(base, CHUNK), :], acc.at[a], sem_g[b])
        pltpu.async_copy(hu_hbm.at[idx_s.at[b]], rows_a.at[b], sem_g[b])
        pltpu.async_copy(hw_hbm.at[idx_t.at[b]], rows_b.at[b], sem_g[b])

    def wait_gathers(k, b, a):
        base = (start + k) * CHUNK
        pltpu.make_async_copy(me_hbm.at[pl.ds(base, CHUNK), :], acc.at[a],
                              sem_g[b]).wait()
        pltpu.make_async_copy(hu_hbm.at[idx_s.at[b]], rows_a.at[b],
                              sem_g[b]).wait()
        pltpu.make_async_copy(hw_hbm.at[idx_t.at[b]], rows_b.at[b],
                              sem_g[b]).wait()

    def do_adds(b, a):
        def row_body(r, rc):
            for j in range(OUT_DIM // LANES):
                sl = pl.ds(j * LANES, LANES)
                plsc.addupdate(acc.at[a, r, sl], rows_a[b, r, sl])
                plsc.addupdate(acc.at[a, r, sl], rows_b[b, r, sl])
            return rc
        lax.fori_loop(0, CHUNK, row_body, 0)

    def issue_out(k, a):
        base = (start + k) * CHUNK
        pltpu.async_copy(acc.at[a], out_hbm.at[pl.ds(base, CHUNK), :],
                         sem_o[a])

    def wait_out(k, a):
        base = (start + k) * CHUNK
        pltpu.make_async_copy(acc.at[a], out_hbm.at[pl.ds(base, CHUNK), :],
                              sem_o[a]).wait()

    issue_idx(0, 0)
    wait_idx(0, 0)
    issue_gathers(0, 0, 0)
    issue_idx(1, 1)

    @pl.loop(0, _N_MAIN, step=_NBUF * _NACC)
    def _steady(k0):
        for i in range(_NBUF * _NACC):
            k = k0 + i
            b = i % _NBUF
            a = i % _NACC
            b1 = (i + 1) % _NBUF
            a1 = (i + 1) % _NACC

            @pl.when(k < _N_MAIN - 1)
            def _prefetch_gathers():
                wait_idx(k + 1, b1)

                @pl.when(k >= 2)
                def _drain_old_out():
                    wait_out(k - 2, a1)

                issue_gathers(k + 1, b1, a1)

            wait_gathers(k, b, a)

            @pl.when(k < _N_MAIN - 2)
            def _prefetch_idx():
                issue_idx(k + 2, b)

            do_adds(b, a)
            issue_out(k, a)

    wait_out(_N_MAIN - 2, (_N_MAIN - 2) % _NACC)
    wait_out(_N_MAIN - 1, (_N_MAIN - 1) % _NACC)

    @pl.when(wid < _N_TAIL)
    def _tail():
        base = (_N_MAIN * _NW + wid) * CHUNK
        pltpu.sync_copy(src_hbm.at[pl.ds(base, CHUNK)], idx_s.at[0])
        pltpu.sync_copy(tgt_hbm.at[pl.ds(base, CHUNK)], idx_t.at[0])
        cp_me = pltpu.async_copy(me_hbm.at[pl.ds(base, CHUNK), :], acc.at[0],
                                 sem_g[0])
        cp_a = pltpu.async_copy(hu_hbm.at[idx_s.at[0]], rows_a.at[0], sem_g[0])
        cp_b = pltpu.async_copy(hw_hbm.at[idx_t.at[0]], rows_b.at[0], sem_g[0])
        cp_me.wait()
        cp_a.wait()
        cp_b.wait()
        do_adds(0, 0)
        pltpu.sync_copy(acc.at[0], out_hbm.at[pl.ds(base, CHUNK), :])


def _sc_combine(hu, hw, src, tgt, me):
    mesh = plsc.VectorSubcoreMesh(core_axis_name="c", subcore_axis_name="s")
    return pl.kernel(
        _sc_body,
        out_type=jax.ShapeDtypeStruct((N_EDGES, OUT_DIM), jnp.float32),
        mesh=mesh,
        scratch_types=[
            pltpu.VMEM((_NBUF, CHUNK), jnp.int32),
            pltpu.VMEM((_NBUF, CHUNK), jnp.int32),
            pltpu.VMEM((_NBUF, CHUNK, OUT_DIM), jnp.float32),
            pltpu.VMEM((_NBUF, CHUNK, OUT_DIM), jnp.float32),
            pltpu.VMEM((_NACC, CHUNK, OUT_DIM), jnp.float32),
            pltpu.SemaphoreType.DMA,
            pltpu.SemaphoreType.DMA,
            pltpu.SemaphoreType.DMA,
            pltpu.SemaphoreType.DMA,
            pltpu.SemaphoreType.DMA,
            pltpu.SemaphoreType.DMA,
            pltpu.SemaphoreType.DMA,
        ],
    )(hu, hw, src, tgt, me)


def kernel(h, edge_index, e, W_e, W_hu, W_hw):
    src = edge_index[0].astype(jnp.int32)
    tgt = edge_index[1].astype(jnp.int32)
    hu, hw = _node_transform(h, W_hu.T, W_hw.T)
    me = _edge_transform(e, W_e.T)
    return _sc_combine(hu, hw, src, tgt, me)


# speedup vs baseline: 1.3795x; 1.3795x over previous
import functools

import jax
import jax.numpy as jnp
from jax import lax
from jax.experimental import pallas as pl
from jax.experimental.pallas import tpu as pltpu
from jax.experimental.pallas import tpu_sc as plsc

N_NODES = 10000
N_EDGES = 320000
IN_DIM = 128
OUT_DIM = 128
EDGE_DIM = 16

LANES = 16
CHUNK = 128
NCHUNKS = N_EDGES // CHUNK

_BN = 1000
_BE = 3200


def _node_body(h_ref, wu_ref, ww_ref, hu_ref, hw_ref):
    h = h_ref[...]
    hu_ref[...] = jnp.dot(h, wu_ref[...], preferred_element_type=jnp.float32)
    hw_ref[...] = jnp.dot(h, ww_ref[...], preferred_element_type=jnp.float32)


def _node_transform(h, wuT, wwT):
    return pl.pallas_call(
        _node_body,
        grid=(N_NODES // _BN,),
        in_specs=[
            pl.BlockSpec((_BN, IN_DIM), lambda i: (i, 0)),
            pl.BlockSpec((IN_DIM, OUT_DIM), lambda i: (0, 0)),
            pl.BlockSpec((IN_DIM, OUT_DIM), lambda i: (0, 0)),
        ],
        out_specs=[
            pl.BlockSpec((_BN, OUT_DIM), lambda i: (i, 0)),
            pl.BlockSpec((_BN, OUT_DIM), lambda i: (i, 0)),
        ],
        out_shape=[
            jax.ShapeDtypeStruct((N_NODES, OUT_DIM), jnp.float32),
            jax.ShapeDtypeStruct((N_NODES, OUT_DIM), jnp.float32),
        ],
    )(h, wuT, wwT)


def _final_body(e_ref, we_ref, g_ref, out_ref):
    out_ref[...] = g_ref[...] + jnp.dot(e_ref[...], we_ref[...],
                                        preferred_element_type=jnp.float32)


def _final_combine(e, weT, gsum):
    return pl.pallas_call(
        _final_body,
        grid=(N_EDGES // _BE,),
        in_specs=[
            pl.BlockSpec((_BE, EDGE_DIM), lambda i: (i, 0)),
            pl.BlockSpec((EDGE_DIM, OUT_DIM), lambda i: (0, 0)),
            pl.BlockSpec((_BE, OUT_DIM), lambda i: (i, 0)),
        ],
        out_specs=pl.BlockSpec((_BE, OUT_DIM), lambda i: (i, 0)),
        out_shape=jax.ShapeDtypeStruct((N_EDGES, OUT_DIM), jnp.float32),
    )(e, weT, gsum)


_NC = 2
_NS = 16
_NW = _NC * _NS


_N_MAIN = NCHUNKS // _NW
_N_TAIL = NCHUNKS - _N_MAIN * _NW
_NBUF = 2
_NACC = 3


def _sc_body(hu_hbm, hw_hbm, src_hbm, tgt_hbm, out_hbm,
             idx_s, idx_t, rows_a, rows_b, acc,
             sem_i0, sem_i1, sem_g0, sem_g1, sem_o0, sem_o1, sem_o2):
    wid = lax.axis_index("s") * _NC + lax.axis_index("c")
    start = wid * _N_MAIN
    sem_i = (sem_i0, sem_i1)
    sem_g = (sem_g0, sem_g1)
    sem_o = (sem_o0, sem_o1, sem_o2)

    def issue_idx(k, b):
        base = (start + k) * CHUNK
        pltpu.async_copy(src_hbm.at[pl.ds(base, CHUNK)], idx_s.at[b], sem_i[b])
        pltpu.async_copy(tgt_hbm.at[pl.ds(base, CHUNK)], idx_t.at[b], sem_i[b])

    def wait_idx(k, b):
        base = (start + k) * CHUNK
        pltpu.make_async_copy(src_hbm.at[pl.ds(base, CHUNK)], idx_s.at[b],
                              sem_i[b]).wait()
        pltpu.make_async_copy(tgt_hbm.at[pl.ds(base, CHUNK)], idx_t.at[b],
                              sem_i[b]).wait()

    def issue_gathers(k, b, a):
        del a
        pltpu.async_copy(hu_hbm.at[idx_s.at[b]], rows_a.at[b], sem_g[b])
        pltpu.async_copy(hw_hbm.at[idx_t.at[b]], rows_b.at[b], sem_g[b])

    def wait_gathers(k, b, a):
        del a
        pltpu.make_async_copy(hu_hbm.at[idx_s.at[b]], rows_a.at[b],
                              sem_g[b]).wait()
        pltpu.make_async_copy(hw_hbm.at[idx_t.at[b]], rows_b.at[b],
                              sem_g[b]).wait()

    def do_adds(b, a):
        def row_body(r, rc):
            for j in range(OUT_DIM // LANES):
                sl = pl.ds(j * LANES, LANES)
                acc[a, r, sl] = rows_a[b, r, sl] + rows_b[b, r, sl]
            return rc
        lax.fori_loop(0, CHUNK, row_body, 0)

    def issue_out(k, a):
        base = (start + k) * CHUNK
        pltpu.async_copy(acc.at[a], out_hbm.at[pl.ds(base, CHUNK), :],
                         sem_o[a])

    def wait_out(k, a):
        base = (start + k) * CHUNK
        pltpu.make_async_copy(acc.at[a], out_hbm.at[pl.ds(base, CHUNK), :],
                              sem_o[a]).wait()

    issue_idx(0, 0)
    wait_idx(0, 0)
    issue_gathers(0, 0, 0)
    issue_idx(1, 1)

    @pl.loop(0, _N_MAIN, step=_NBUF * _NACC)
    def _steady(k0):
        for i in range(_NBUF * _NACC):
            k = k0 + i
            b = i % _NBUF
            a = i % _NACC
            b1 = (i + 1) % _NBUF
            a1 = (i + 1) % _NACC

            @pl.when(k < _N_MAIN - 1)
            def _prefetch_gathers():
                wait_idx(k + 1, b1)
                issue_gathers(k + 1, b1, a1)

            wait_gathers(k, b, a)

            @pl.when(k < _N_MAIN - 2)
            def _prefetch_idx():
                issue_idx(k + 2, b)

            @pl.when(k >= _NACC)
            def _drain_old_out():
                wait_out(k - _NACC, a)

            do_adds(b, a)
            issue_out(k, a)

    for j in range(_NACC):
        kk = _N_MAIN - _NACC + j
        wait_out(kk, kk % _NACC)

    @pl.when(wid < _N_TAIL)
    def _tail():
        base = (_N_MAIN * _NW + wid) * CHUNK
        pltpu.sync_copy(src_hbm.at[pl.ds(base, CHUNK)], idx_s.at[0])
        pltpu.sync_copy(tgt_hbm.at[pl.ds(base, CHUNK)], idx_t.at[0])
        cp_a = pltpu.async_copy(hu_hbm.at[idx_s.at[0]], rows_a.at[0], sem_g[0])
        cp_b = pltpu.async_copy(hw_hbm.at[idx_t.at[0]], rows_b.at[0], sem_g[0])
        cp_a.wait()
        cp_b.wait()
        do_adds(0, 0)
        pltpu.sync_copy(acc.at[0], out_hbm.at[pl.ds(base, CHUNK), :])


def _sc_gsum(hu, hw, src, tgt):
    mesh = plsc.VectorSubcoreMesh(core_axis_name="c", subcore_axis_name="s")
    return pl.kernel(
        _sc_body,
        out_type=jax.ShapeDtypeStruct((N_EDGES, OUT_DIM), jnp.float32),
        mesh=mesh,
        scratch_types=[
            pltpu.VMEM((_NBUF, CHUNK), jnp.int32),
            pltpu.VMEM((_NBUF, CHUNK), jnp.int32),
            pltpu.VMEM((_NBUF, CHUNK, OUT_DIM), jnp.float32),
            pltpu.VMEM((_NBUF, CHUNK, OUT_DIM), jnp.float32),
            pltpu.VMEM((_NACC, CHUNK, OUT_DIM), jnp.float32),
            pltpu.SemaphoreType.DMA,
            pltpu.SemaphoreType.DMA,
            pltpu.SemaphoreType.DMA,
            pltpu.SemaphoreType.DMA,
            pltpu.SemaphoreType.DMA,
            pltpu.SemaphoreType.DMA,
            pltpu.SemaphoreType.DMA,
        ],
    )(hu, hw, src, tgt)


def kernel(h, edge_index, e, W_e, W_hu, W_hw):
    src = edge_index[0].astype(jnp.int32)
    tgt = edge_index[1].astype(jnp.int32)
    hu, hw = _node_transform(h, W_hu.T, W_hw.T)
    gsum = _sc_gsum(hu, hw, src, tgt)
    return _final_combine(e, W_e.T, gsum)


# speedup vs baseline: 1.3876x; 1.0059x over previous
import functools

import jax
import jax.numpy as jnp
from jax import lax
from jax.experimental import pallas as pl
from jax.experimental.pallas import tpu as pltpu
from jax.experimental.pallas import tpu_sc as plsc

N_NODES = 10000
N_EDGES = 320000
IN_DIM = 128
OUT_DIM = 128
EDGE_DIM = 16

LANES = 16
CHUNK = 80
NCHUNKS = N_EDGES // CHUNK

_BN = 1000
_BE = 3200


def _node_body(h_ref, wu_ref, ww_ref, hu_ref, hw_ref):
    h = h_ref[...]
    hu_ref[...] = jnp.dot(h, wu_ref[...], preferred_element_type=jnp.float32)
    hw_ref[...] = jnp.dot(h, ww_ref[...], preferred_element_type=jnp.float32)


def _node_transform(h, wuT, wwT):
    return pl.pallas_call(
        _node_body,
        grid=(N_NODES // _BN,),
        in_specs=[
            pl.BlockSpec((_BN, IN_DIM), lambda i: (i, 0)),
            pl.BlockSpec((IN_DIM, OUT_DIM), lambda i: (0, 0)),
            pl.BlockSpec((IN_DIM, OUT_DIM), lambda i: (0, 0)),
        ],
        out_specs=[
            pl.BlockSpec((_BN, OUT_DIM), lambda i: (i, 0)),
            pl.BlockSpec((_BN, OUT_DIM), lambda i: (i, 0)),
        ],
        out_shape=[
            jax.ShapeDtypeStruct((N_NODES, OUT_DIM), jnp.float32),
            jax.ShapeDtypeStruct((N_NODES, OUT_DIM), jnp.float32),
        ],
    )(h, wuT, wwT)


def _final_body(e_ref, we_ref, g_ref, out_ref):
    out_ref[...] = g_ref[...] + jnp.dot(e_ref[...], we_ref[...],
                                        preferred_element_type=jnp.float32)


def _final_combine(e, weT, gsum):
    return pl.pallas_call(
        _final_body,
        grid=(N_EDGES // _BE,),
        in_specs=[
            pl.BlockSpec((_BE, EDGE_DIM), lambda i: (i, 0)),
            pl.BlockSpec((EDGE_DIM, OUT_DIM), lambda i: (0, 0)),
            pl.BlockSpec((_BE, OUT_DIM), lambda i: (i, 0)),
        ],
        out_specs=pl.BlockSpec((_BE, OUT_DIM), lambda i: (i, 0)),
        out_shape=jax.ShapeDtypeStruct((N_EDGES, OUT_DIM), jnp.float32),
    )(e, weT, gsum)


_NC = 2
_NS = 16
_NW = _NC * _NS


_N_MAIN = NCHUNKS // _NW
_NBUF = 4
_NACC = 3
_UNROLL = 12
_N_STEADY = (_N_MAIN // _UNROLL) * _UNROLL


def _sc_body(hu_hbm, hw_hbm, src_hbm, tgt_hbm, out_hbm,
             idx_s, idx_t, rows_a, rows_b, acc,
             sem_i0, sem_i1, sem_i2, sem_i3,
             sem_g0, sem_g1, sem_g2, sem_g3,
             sem_o0, sem_o1, sem_o2):
    wid = lax.axis_index("s") * _NC + lax.axis_index("c")
    start = wid * _N_MAIN
    sem_i = (sem_i0, sem_i1, sem_i2, sem_i3)
    sem_g = (sem_g0, sem_g1, sem_g2, sem_g3)
    sem_o = (sem_o0, sem_o1, sem_o2)

    def issue_idx(k, b):
        base = (start + k) * CHUNK
        pltpu.async_copy(src_hbm.at[pl.ds(base, CHUNK)], idx_s.at[b], sem_i[b])
        pltpu.async_copy(tgt_hbm.at[pl.ds(base, CHUNK)], idx_t.at[b], sem_i[b])

    def wait_idx(k, b):
        base = (start + k) * CHUNK
        pltpu.make_async_copy(src_hbm.at[pl.ds(base, CHUNK)], idx_s.at[b],
                              sem_i[b]).wait()
        pltpu.make_async_copy(tgt_hbm.at[pl.ds(base, CHUNK)], idx_t.at[b],
                              sem_i[b]).wait()

    def issue_gathers(k, b, a):
        del a
        pltpu.async_copy(hu_hbm.at[idx_s.at[b]], rows_a.at[b], sem_g[b])
        pltpu.async_copy(hw_hbm.at[idx_t.at[b]], rows_b.at[b], sem_g[b])

    def wait_gathers(k, b, a):
        del a
        pltpu.make_async_copy(hu_hbm.at[idx_s.at[b]], rows_a.at[b],
                              sem_g[b]).wait()
        pltpu.make_async_copy(hw_hbm.at[idx_t.at[b]], rows_b.at[b],
                              sem_g[b]).wait()

    def do_adds(b, a):
        def row_body(r, rc):
            for j in range(OUT_DIM // LANES):
                sl = pl.ds(j * LANES, LANES)
                acc[a, r, sl] = rows_a[b, r, sl] + rows_b[b, r, sl]
            return rc
        lax.fori_loop(0, CHUNK, row_body, 0)

    def issue_out(k, a):
        base = (start + k) * CHUNK
        pltpu.async_copy(acc.at[a], out_hbm.at[pl.ds(base, CHUNK), :],
                         sem_o[a])

    def wait_out(k, a):
        base = (start + k) * CHUNK
        pltpu.make_async_copy(acc.at[a], out_hbm.at[pl.ds(base, CHUNK), :],
                              sem_o[a]).wait()

    def iteration(k, i):
        b = i % _NBUF
        a = i % _NACC
        b3 = (i + 3) % _NBUF

        @pl.when(k < _N_MAIN - 3)
        def _prefetch_gathers():
            wait_idx(k + 3, b3)
            issue_gathers(k + 3, b3, 0)

        wait_gathers(k, b, a)

        @pl.when(k < _N_MAIN - 4)
        def _prefetch_idx():
            issue_idx(k + 4, b)

        @pl.when(k >= _NACC)
        def _drain_old_out():
            wait_out(k - _NACC, a)

        do_adds(b, a)
        issue_out(k, a)

    for j in range(3):
        issue_idx(j, j)
    for j in range(3):
        wait_idx(j, j)
        issue_gathers(j, j, 0)
    issue_idx(3, 3)

    @pl.loop(0, _N_STEADY, step=_UNROLL)
    def _steady(k0):
        for i in range(_UNROLL):
            iteration(k0 + i, i)

    for j in range(_N_MAIN - _N_STEADY):
        iteration(_N_STEADY + j, j)

    for j in range(_NACC):
        kk = _N_MAIN - _NACC + j
        wait_out(kk, kk % _NACC)


def _sc_gsum(hu, hw, src, tgt):
    mesh = plsc.VectorSubcoreMesh(core_axis_name="c", subcore_axis_name="s")
    return pl.kernel(
        _sc_body,
        out_type=jax.ShapeDtypeStruct((N_EDGES, OUT_DIM), jnp.float32),
        mesh=mesh,
        scratch_types=[
            pltpu.VMEM((_NBUF, CHUNK), jnp.int32),
            pltpu.VMEM((_NBUF, CHUNK), jnp.int32),
            pltpu.VMEM((_NBUF, CHUNK, OUT_DIM), jnp.float32),
            pltpu.VMEM((_NBUF, CHUNK, OUT_DIM), jnp.float32),
            pltpu.VMEM((_NACC, CHUNK, OUT_DIM), jnp.float32),
            pltpu.SemaphoreType.DMA,
            pltpu.SemaphoreType.DMA,
            pltpu.SemaphoreType.DMA,
            pltpu.SemaphoreType.DMA,
            pltpu.SemaphoreType.DMA,
            pltpu.SemaphoreType.DMA,
            pltpu.SemaphoreType.DMA,
            pltpu.SemaphoreType.DMA,
            pltpu.SemaphoreType.DMA,
            pltpu.SemaphoreType.DMA,
            pltpu.SemaphoreType.DMA,
        ],
    )(hu, hw, src, tgt)


def kernel(h, edge_index, e, W_e, W_hu, W_hw):
    src = edge_index[0].astype(jnp.int32)
    tgt = edge_index[1].astype(jnp.int32)
    hu, hw = _node_transform(h, W_hu.T, W_hw.T)
    gsum = _sc_gsum(hu, hw, src, tgt)
    return _final_combine(e, W_e.T, gsum)


# speedup vs baseline: 1.4248x; 1.0269x over previous
import functools

import jax
import jax.numpy as jnp
from jax import lax
from jax.experimental import pallas as pl
from jax.experimental.pallas import tpu as pltpu
from jax.experimental.pallas import tpu_sc as plsc

N_NODES = 10000
N_EDGES = 320000
IN_DIM = 128
OUT_DIM = 128
EDGE_DIM = 16

LANES = 16
CHUNK = 80
NCHUNKS = N_EDGES // CHUNK

_BN = 1000
_BE = 3200


def _node_body(h_ref, wu_ref, ww_ref, hu_ref, hw_ref):
    h = h_ref[...]
    hu_ref[...] = jnp.dot(h, wu_ref[...], preferred_element_type=jnp.float32)
    hw_ref[...] = jnp.dot(h, ww_ref[...], preferred_element_type=jnp.float32)


def _node_transform(h, wuT, wwT):
    return pl.pallas_call(
        _node_body,
        grid=(N_NODES // _BN,),
        in_specs=[
            pl.BlockSpec((_BN, IN_DIM), lambda i: (i, 0)),
            pl.BlockSpec((IN_DIM, OUT_DIM), lambda i: (0, 0)),
            pl.BlockSpec((IN_DIM, OUT_DIM), lambda i: (0, 0)),
        ],
        out_specs=[
            pl.BlockSpec((_BN, OUT_DIM), lambda i: (i, 0)),
            pl.BlockSpec((_BN, OUT_DIM), lambda i: (i, 0)),
        ],
        out_shape=[
            jax.ShapeDtypeStruct((N_NODES, OUT_DIM), jnp.float32),
            jax.ShapeDtypeStruct((N_NODES, OUT_DIM), jnp.float32),
        ],
    )(h, wuT, wwT)


def _final_body(e_ref, we_ref, g_ref, out_ref):
    out_ref[...] = g_ref[...] + jnp.dot(e_ref[...], we_ref[...],
                                        preferred_element_type=jnp.float32)


def _final_combine(e, weT, gsum):
    return pl.pallas_call(
        _final_body,
        grid=(N_EDGES // _BE,),
        in_specs=[
            pl.BlockSpec((_BE, EDGE_DIM), lambda i: (i, 0)),
            pl.BlockSpec((EDGE_DIM, OUT_DIM), lambda i: (0, 0)),
            pl.BlockSpec((_BE, OUT_DIM), lambda i: (i, 0)),
        ],
        out_specs=pl.BlockSpec((_BE, OUT_DIM), lambda i: (i, 0)),
        out_shape=jax.ShapeDtypeStruct((N_EDGES, OUT_DIM), jnp.float32),
    )(e, weT, gsum)


_NC = 2
_NS = 16
_NW = _NC * _NS


_N_MAIN = NCHUNKS // _NW
_NBUF = 4
_NACC = 3
_UNROLL = 12
_N_STEADY = (_N_MAIN // _UNROLL) * _UNROLL


def _sc_body(hu_hbm, hw_hbm, ei_hbm, out_hbm,
             idx_s, idx_t, rows_a, rows_b, acc,
             sem_i0, sem_i1, sem_i2, sem_i3,
             sem_g0, sem_g1, sem_g2, sem_g3,
             sem_o0, sem_o1, sem_o2):
    wid = lax.axis_index("s") * _NC + lax.axis_index("c")
    start = wid * _N_MAIN
    sem_i = (sem_i0, sem_i1, sem_i2, sem_i3)
    sem_g = (sem_g0, sem_g1, sem_g2, sem_g3)
    sem_o = (sem_o0, sem_o1, sem_o2)

    def issue_idx(k, b):
        base = (start + k) * CHUNK
        pltpu.async_copy(ei_hbm.at[pl.ds(base, CHUNK)], idx_s.at[b], sem_i[b])
        pltpu.async_copy(ei_hbm.at[pl.ds(N_EDGES + base, CHUNK)], idx_t.at[b],
                         sem_i[b])

    def wait_idx(k, b):
        base = (start + k) * CHUNK
        pltpu.make_async_copy(ei_hbm.at[pl.ds(base, CHUNK)], idx_s.at[b],
                              sem_i[b]).wait()
        pltpu.make_async_copy(ei_hbm.at[pl.ds(N_EDGES + base, CHUNK)],
                              idx_t.at[b], sem_i[b]).wait()

    def issue_gathers(k, b, a):
        del a
        pltpu.async_copy(hu_hbm.at[idx_s.at[b]], rows_a.at[b], sem_g[b])
        pltpu.async_copy(hw_hbm.at[idx_t.at[b]], rows_b.at[b], sem_g[b])

    def wait_gathers(k, b, a):
        del a
        pltpu.make_async_copy(hu_hbm.at[idx_s.at[b]], rows_a.at[b],
                              sem_g[b]).wait()
        pltpu.make_async_copy(hw_hbm.at[idx_t.at[b]], rows_b.at[b],
                              sem_g[b]).wait()

    def do_adds(b, a):
        def row_body(r, rc):
            for j in range(OUT_DIM // LANES):
                sl = pl.ds(j * LANES, LANES)
                acc[a, r, sl] = rows_a[b, r, sl] + rows_b[b, r, sl]
            return rc
        lax.fori_loop(0, CHUNK, row_body, 0)

    def issue_out(k, a):
        base = (start + k) * CHUNK
        pltpu.async_copy(acc.at[a], out_hbm.at[pl.ds(base, CHUNK), :],
                         sem_o[a])

    def wait_out(k, a):
        base = (start + k) * CHUNK
        pltpu.make_async_copy(acc.at[a], out_hbm.at[pl.ds(base, CHUNK), :],
                              sem_o[a]).wait()

    def iteration(k, i):
        b = i % _NBUF
        a = i % _NACC
        b3 = (i + 3) % _NBUF

        @pl.when(k < _N_MAIN - 3)
        def _prefetch_gathers():
            wait_idx(k + 3, b3)
            issue_gathers(k + 3, b3, 0)

        wait_gathers(k, b, a)

        @pl.when(k < _N_MAIN - 4)
        def _prefetch_idx():
            issue_idx(k + 4, b)

        @pl.when(k >= _NACC)
        def _drain_old_out():
            wait_out(k - _NACC, a)

        do_adds(b, a)
        issue_out(k, a)

    for j in range(3):
        issue_idx(j, j)
    for j in range(3):
        wait_idx(j, j)
        issue_gathers(j, j, 0)
    issue_idx(3, 3)

    @pl.loop(0, _N_STEADY, step=_UNROLL)
    def _steady(k0):
        for i in range(_UNROLL):
            iteration(k0 + i, i)

    for j in range(_N_MAIN - _N_STEADY):
        iteration(_N_STEADY + j, j)

    for j in range(_NACC):
        kk = _N_MAIN - _NACC + j
        wait_out(kk, kk % _NACC)


def _sc_gsum(hu, hw, edge_index):
    mesh = plsc.VectorSubcoreMesh(core_axis_name="c", subcore_axis_name="s")
    return pl.kernel(
        _sc_body,
        out_type=jax.ShapeDtypeStruct((N_EDGES, OUT_DIM), jnp.float32),
        mesh=mesh,
        scratch_types=[
            pltpu.VMEM((_NBUF, CHUNK), jnp.int32),
            pltpu.VMEM((_NBUF, CHUNK), jnp.int32),
            pltpu.VMEM((_NBUF, CHUNK, OUT_DIM), jnp.float32),
            pltpu.VMEM((_NBUF, CHUNK, OUT_DIM), jnp.float32),
            pltpu.VMEM((_NACC, CHUNK, OUT_DIM), jnp.float32),
            pltpu.SemaphoreType.DMA,
            pltpu.SemaphoreType.DMA,
            pltpu.SemaphoreType.DMA,
            pltpu.SemaphoreType.DMA,
            pltpu.SemaphoreType.DMA,
            pltpu.SemaphoreType.DMA,
            pltpu.SemaphoreType.DMA,
            pltpu.SemaphoreType.DMA,
            pltpu.SemaphoreType.DMA,
            pltpu.SemaphoreType.DMA,
            pltpu.SemaphoreType.DMA,
        ],
    )(hu, hw, edge_index)


def kernel(h, edge_index, e, W_e, W_hu, W_hw):
    hu, hw = _node_transform(h, W_hu.T, W_hw.T)
    gsum = _sc_gsum(hu, hw, edge_index.astype(jnp.int32).reshape(-1))
    return _final_combine(e, W_e.T, gsum)


# speedup vs baseline: 1.4497x; 1.0174x over previous
import functools

import jax
import jax.numpy as jnp
from jax import lax
from jax.experimental import pallas as pl
from jax.experimental.pallas import tpu as pltpu
from jax.experimental.pallas import tpu_sc as plsc

N_NODES = 10000
N_EDGES = 320000
IN_DIM = 128
OUT_DIM = 128
EDGE_DIM = 16

LANES = 16
CHUNK = 80
NCHUNKS = N_EDGES // CHUNK

_BN = 1000
_BE = 3200


def _node_body(h_ref, wu_ref, ww_ref, hu_ref, hw_ref):
    h = h_ref[...]
    hu_ref[...] = jnp.dot(h, wu_ref[...], preferred_element_type=jnp.float32)
    hw_ref[...] = jnp.dot(h, ww_ref[...], preferred_element_type=jnp.float32)


def _node_transform(h, wuT, wwT):
    return pl.pallas_call(
        _node_body,
        grid=(N_NODES // _BN,),
        in_specs=[
            pl.BlockSpec((_BN, IN_DIM), lambda i: (i, 0)),
            pl.BlockSpec((IN_DIM, OUT_DIM), lambda i: (0, 0)),
            pl.BlockSpec((IN_DIM, OUT_DIM), lambda i: (0, 0)),
        ],
        out_specs=[
            pl.BlockSpec((_BN, OUT_DIM), lambda i: (i, 0)),
            pl.BlockSpec((_BN, OUT_DIM), lambda i: (i, 0)),
        ],
        out_shape=[
            jax.ShapeDtypeStruct((N_NODES, OUT_DIM), jnp.float32),
            jax.ShapeDtypeStruct((N_NODES, OUT_DIM), jnp.float32),
        ],
    )(h, wuT, wwT)


_EPR = 128 // EDGE_DIM


def _final_body(e2_ref, wb_ref, g_ref, out_ref):
    prod = jnp.dot(e2_ref[...], wb_ref[...], preferred_element_type=jnp.float32)
    out_ref[...] = g_ref[...] + prod.reshape(_BE, OUT_DIM)


def _final_combine(e, weT, gsum):
    e2 = e.reshape(N_EDGES // _EPR, EDGE_DIM * _EPR)
    wb = jnp.kron(jnp.eye(_EPR, dtype=weT.dtype), weT)
    return pl.pallas_call(
        _final_body,
        grid=(N_EDGES // _BE,),
        in_specs=[
            pl.BlockSpec((_BE // _EPR, EDGE_DIM * _EPR), lambda i: (i, 0)),
            pl.BlockSpec((EDGE_DIM * _EPR, OUT_DIM * _EPR), lambda i: (0, 0)),
            pl.BlockSpec((_BE, OUT_DIM), lambda i: (i, 0)),
        ],
        out_specs=pl.BlockSpec((_BE, OUT_DIM), lambda i: (i, 0)),
        out_shape=jax.ShapeDtypeStruct((N_EDGES, OUT_DIM), jnp.float32),
    )(e2, wb, gsum)


_NC = 2
_NS = 16
_NW = _NC * _NS


_N_MAIN = NCHUNKS // _NW
_NBUF = 4
_NACC = 3
_UNROLL = 12
_N_STEADY = (_N_MAIN // _UNROLL) * _UNROLL


def _sc_body(hu_hbm, hw_hbm, ei_hbm, out_hbm,
             idx_s, idx_t, rows_a, rows_b, acc,
             sem_i0, sem_i1, sem_i2, sem_i3,
             sem_g0, sem_g1, sem_g2, sem_g3,
             sem_o0, sem_o1, sem_o2):
    wid = lax.axis_index("s") * _NC + lax.axis_index("c")
    start = wid * _N_MAIN
    sem_i = (sem_i0, sem_i1, sem_i2, sem_i3)
    sem_g = (sem_g0, sem_g1, sem_g2, sem_g3)
    sem_o = (sem_o0, sem_o1, sem_o2)

    def issue_idx(k, b):
        base = (start + k) * CHUNK
        pltpu.async_copy(ei_hbm.at[pl.ds(base, CHUNK)], idx_s.at[b], sem_i[b])
        pltpu.async_copy(ei_hbm.at[pl.ds(N_EDGES + base, CHUNK)], idx_t.at[b],
                         sem_i[b])

    def wait_idx(k, b):
        base = (start + k) * CHUNK
        pltpu.make_async_copy(ei_hbm.at[pl.ds(base, CHUNK)], idx_s.at[b],
                              sem_i[b]).wait()
        pltpu.make_async_copy(ei_hbm.at[pl.ds(N_EDGES + base, CHUNK)],
                              idx_t.at[b], sem_i[b]).wait()

    def issue_gathers(k, b, a):
        del a
        pltpu.async_copy(hu_hbm.at[idx_s.at[b]], rows_a.at[b], sem_g[b])
        pltpu.async_copy(hw_hbm.at[idx_t.at[b]], rows_b.at[b], sem_g[b])

    def wait_gathers(k, b, a):
        del a
        pltpu.make_async_copy(hu_hbm.at[idx_s.at[b]], rows_a.at[b],
                              sem_g[b]).wait()
        pltpu.make_async_copy(hw_hbm.at[idx_t.at[b]], rows_b.at[b],
                              sem_g[b]).wait()

    def do_adds(b, a):
        def row_body(r, rc):
            for j in range(OUT_DIM // LANES):
                sl = pl.ds(j * LANES, LANES)
                acc[a, r, sl] = rows_a[b, r, sl] + rows_b[b, r, sl]
            return rc
        lax.fori_loop(0, CHUNK, row_body, 0)

    def issue_out(k, a):
        base = (start + k) * CHUNK
        pltpu.async_copy(acc.at[a], out_hbm.at[pl.ds(base, CHUNK), :],
                         sem_o[a])

    def wait_out(k, a):
        base = (start + k) * CHUNK
        pltpu.make_async_copy(acc.at[a], out_hbm.at[pl.ds(base, CHUNK), :],
                              sem_o[a]).wait()

    def iteration(k, i):
        b = i % _NBUF
        a = i % _NACC
        b3 = (i + 3) % _NBUF

        @pl.when(k < _N_MAIN - 3)
        def _prefetch_gathers():
            wait_idx(k + 3, b3)
            issue_gathers(k + 3, b3, 0)

        wait_gathers(k, b, a)

        @pl.when(k < _N_MAIN - 4)
        def _prefetch_idx():
            issue_idx(k + 4, b)

        @pl.when(k >= _NACC)
        def _drain_old_out():
            wait_out(k - _NACC, a)

        do_adds(b, a)
        issue_out(k, a)

    for j in range(3):
        issue_idx(j, j)
    for j in range(3):
        wait_idx(j, j)
        issue_gathers(j, j, 0)
    issue_idx(3, 3)

    @pl.loop(0, _N_STEADY, step=_UNROLL)
    def _steady(k0):
        for i in range(_UNROLL):
            iteration(k0 + i, i)

    for j in range(_N_MAIN - _N_STEADY):
        iteration(_N_STEADY + j, j)

    for j in range(_NACC):
        kk = _N_MAIN - _NACC + j
        wait_out(kk, kk % _NACC)


def _sc_gsum(hu, hw, edge_index):
    mesh = plsc.VectorSubcoreMesh(core_axis_name="c", subcore_axis_name="s")
    return pl.kernel(
        _sc_body,
        out_type=jax.ShapeDtypeStruct((N_EDGES, OUT_DIM), jnp.float32),
        mesh=mesh,
        scratch_types=[
            pltpu.VMEM((_NBUF, CHUNK), jnp.int32),
            pltpu.VMEM((_NBUF, CHUNK), jnp.int32),
            pltpu.VMEM((_NBUF, CHUNK, OUT_DIM), jnp.float32),
            pltpu.VMEM((_NBUF, CHUNK, OUT_DIM), jnp.float32),
            pltpu.VMEM((_NACC, CHUNK, OUT_DIM), jnp.float32),
            pltpu.SemaphoreType.DMA,
            pltpu.SemaphoreType.DMA,
            pltpu.SemaphoreType.DMA,
            pltpu.SemaphoreType.DMA,
            pltpu.SemaphoreType.DMA,
            pltpu.SemaphoreType.DMA,
            pltpu.SemaphoreType.DMA,
            pltpu.SemaphoreType.DMA,
            pltpu.SemaphoreType.DMA,
            pltpu.SemaphoreType.DMA,
            pltpu.SemaphoreType.DMA,
        ],
    )(hu, hw, edge_index)


def kernel(h, edge_index, e, W_e, W_hu, W_hw):
    hu, hw = _node_transform(h, W_hu.T, W_hw.T)
    gsum = _sc_gsum(hu, hw, edge_index.astype(jnp.int32).reshape(-1))
    return _final_combine(e, W_e.T, gsum)
